# Initial kernel scaffold; baseline (speedup 1.0000x reference)
import jax
import jax.numpy as jnp
from jax.experimental import pallas as pl


def kernel(x, roots):
    raise NotImplementedError("write your pallas kernel here")


# speedup vs baseline: 4.2728x; 4.2728x over previous
import functools

import jax
import jax.numpy as jnp
from jax.experimental import pallas as pl


def _quantize_one(res, row):
    a = jnp.abs(res)
    neg = (res < 0.0)
    negi = neg.astype(jnp.int32)

    m1 = jnp.max(a, axis=0)
    i1 = jnp.min(jnp.where(a == m1[None, :], row, 8), axis=0)
    mask1 = row == i1[None, :]
    a2 = jnp.where(mask1, -1.0, a)
    m2 = jnp.max(a2, axis=0)
    i2 = jnp.min(jnp.where(a2 == m2[None, :], row, 8), axis=0)
    score1 = m1 + m2

    total = jnp.sum(a, axis=0)
    mn = jnp.min(a, axis=0)
    tied = a == mn[None, :]
    tiedneg = tied & neg
    has_tn = jnp.max(jnp.where(tiedneg, 1, 0), axis=0) == 1
    k_neg = jnp.max(jnp.where(tiedneg, row, -1), axis=0)
    k_pos = jnp.min(jnp.where(tied, row, 8), axis=0)
    imn = jnp.where(has_tn, k_neg, k_pos)
    parity_odd = (jnp.sum(negi, axis=0) & 1) == 1
    score2 = 0.5 * total - jnp.where(parity_odd, mn, 0.0)

    use1 = score1 >= score2

    lo = jnp.minimum(i1, i2)
    hi = jnp.maximum(i1, i2)
    pair = lo * (15 - lo) // 2 + (hi - lo - 1)
    s_lo = jnp.sum(jnp.where(row == lo[None, :], negi, 0), axis=0)
    s_hi = jnp.sum(jnp.where(row == hi[None, :], negi, 0), axis=0)
    index1 = 4 * pair + 2 * s_lo + s_hi

    flip = parity_odd[None, :] & (row == imn[None, :])
    bits = jnp.where(flip, 1 - negi, negi)
    b = jnp.sum(bits << row, axis=0)
    index2 = 112 + (b >> 1)

    idx = jnp.where(use1, index1, index2)

    sgnval = jnp.where(neg, -1.0, 1.0)
    y_t1 = jnp.where(mask1 | (row == i2[None, :]), sgnval, 0.0)
    y_t2 = jnp.where(bits == 1, -0.5, 0.5)
    y = jnp.where(use1[None, :], y_t1, y_t2)
    return y, idx


def _bf16(v):
    return v.astype(jnp.bfloat16).astype(jnp.float32)


def _body(x_ref, q_ref, i1_ref, i2_ref, err_ref):
    xt = x_ref[...]
    row = jax.lax.broadcasted_iota(jnp.int32, xt.shape, 0)

    y1, idx1 = _quantize_one(_bf16(xt), row)
    r1 = xt - y1
    y2, idx2 = _quantize_one(_bf16(r1), row)
    q = y1 + y2

    q_ref[...] = q
    i1_ref[0, 0, :] = idx1
    i2_ref[0, 0, :] = idx2

    resid = xt - q
    partial = jnp.sum(resid * resid).reshape(1, 1)

    @pl.when(pl.program_id(0) == 0)
    def _init():
        err_ref[...] = partial

    @pl.when(pl.program_id(0) != 0)
    def _acc():
        err_ref[...] += partial


@functools.partial(jax.jit, static_argnames=("interpret",))
def kernel(x, roots, interpret=False):
    del roots
    orig_shape = x.shape
    n = x.shape[0] * x.shape[1]
    L = 8192
    g = n // L

    xt = x.reshape(n, 8).T

    q_t, idx1, idx2, err_sum = pl.pallas_call(
        _body,
        grid=(g,),
        in_specs=[pl.BlockSpec((8, L), lambda i: (0, i))],
        out_specs=(
            pl.BlockSpec((8, L), lambda i: (0, i)),
            pl.BlockSpec((1, 1, L), lambda i: (i, 0, 0)),
            pl.BlockSpec((1, 1, L), lambda i: (i, 0, 0)),
            pl.BlockSpec((1, 1), lambda i: (0, 0)),
        ),
        out_shape=(
            jax.ShapeDtypeStruct((8, n), jnp.float32),
            jax.ShapeDtypeStruct((g, 1, L), jnp.int32),
            jax.ShapeDtypeStruct((g, 1, L), jnp.int32),
            jax.ShapeDtypeStruct((1, 1), jnp.float32),
        ),
        interpret=interpret,
    )(xt)

    quantized = q_t.T.reshape(orig_shape)
    i1 = idx1.reshape(orig_shape[:-1])
    i2 = idx2.reshape(orig_shape[:-1])
    err = err_sum[0, 0] / jnp.float32(n * 8)
    return (quantized, i1, i2, err)


# speedup vs baseline: 4.5375x; 1.0620x over previous
import functools

import jax
import jax.numpy as jnp
from jax.experimental import pallas as pl

_W = 512


def _quantize_one(res, row):
    a = jnp.abs(res)
    neg = (res < 0.0)
    negi = neg.astype(jnp.int32)

    m1 = jnp.max(a, axis=0)
    i1 = jnp.min(jnp.where(a == m1[None, :], row, 8), axis=0)
    mask1 = row == i1[None, :]
    a2 = jnp.where(mask1, -1.0, a)
    m2 = jnp.max(a2, axis=0)
    i2 = jnp.min(jnp.where(a2 == m2[None, :], row, 8), axis=0)
    score1 = m1 + m2

    total = jnp.sum(a, axis=0)
    mn = jnp.min(a, axis=0)
    tied = a == mn[None, :]
    tiedneg = tied & neg
    has_tn = jnp.max(jnp.where(tiedneg, 1, 0), axis=0) == 1
    k_neg = jnp.max(jnp.where(tiedneg, row, -1), axis=0)
    k_pos = jnp.min(jnp.where(tied, row, 8), axis=0)
    imn = jnp.where(has_tn, k_neg, k_pos)
    parity_odd = (jnp.sum(negi, axis=0) & 1) == 1
    score2 = 0.5 * total - jnp.where(parity_odd, mn, 0.0)

    use1 = score1 >= score2

    lo = jnp.minimum(i1, i2)
    hi = jnp.maximum(i1, i2)
    pair = lo * (15 - lo) // 2 + (hi - lo - 1)
    s_lo = jnp.sum(jnp.where(row == lo[None, :], negi, 0), axis=0)
    s_hi = jnp.sum(jnp.where(row == hi[None, :], negi, 0), axis=0)
    index1 = 4 * pair + 2 * s_lo + s_hi

    flip = parity_odd[None, :] & (row == imn[None, :])
    bits = jnp.where(flip, 1 - negi, negi)
    b = jnp.sum(bits << row, axis=0)
    index2 = 112 + (b >> 1)

    idx = jnp.where(use1, index1, index2)

    sgnval = jnp.where(neg, -1.0, 1.0)
    y_t1 = jnp.where(mask1 | (row == i2[None, :]), sgnval, 0.0)
    y_t2 = jnp.where(bits == 1, -0.5, 0.5)
    y = jnp.where(use1[None, :], y_t1, y_t2)
    return y, idx


def _bf16(v):
    return v.astype(jnp.bfloat16).astype(jnp.float32)


def _body(x_ref, q_ref, i1_ref, i2_ref, err_ref):
    L = x_ref.shape[1]
    row = jax.lax.broadcasted_iota(jnp.int32, (8, _W), 0)
    err_acc = jnp.zeros((1, 1), jnp.float32)

    for c in range(L // _W):
        sl = pl.ds(c * _W, _W)
        xt = x_ref[:, sl]
        y1, idx1 = _quantize_one(_bf16(xt), row)
        r1 = xt - y1
        y2, idx2 = _quantize_one(_bf16(r1), row)
        q = y1 + y2
        q_ref[:, sl] = q
        i1_ref[0, 0, sl] = idx1
        i2_ref[0, 0, sl] = idx2
        resid = xt - q
        err_acc = err_acc + jnp.sum(resid * resid).reshape(1, 1)

    @pl.when(pl.program_id(0) == 0)
    def _init():
        err_ref[...] = err_acc

    @pl.when(pl.program_id(0) != 0)
    def _acc():
        err_ref[...] += err_acc


@functools.partial(jax.jit, static_argnames=("interpret",))
def kernel(x, roots, interpret=False):
    del roots
    orig_shape = x.shape
    n = x.shape[0] * x.shape[1]
    L = 8192
    g = n // L

    xt = x.reshape(n, 8).T

    q_t, idx1, idx2, err_sum = pl.pallas_call(
        _body,
        grid=(g,),
        in_specs=[pl.BlockSpec((8, L), lambda i: (0, i))],
        out_specs=(
            pl.BlockSpec((8, L), lambda i: (0, i)),
            pl.BlockSpec((1, 1, L), lambda i: (i, 0, 0)),
            pl.BlockSpec((1, 1, L), lambda i: (i, 0, 0)),
            pl.BlockSpec((1, 1), lambda i: (0, 0)),
        ),
        out_shape=(
            jax.ShapeDtypeStruct((8, n), jnp.float32),
            jax.ShapeDtypeStruct((g, 1, L), jnp.int32),
            jax.ShapeDtypeStruct((g, 1, L), jnp.int32),
            jax.ShapeDtypeStruct((1, 1), jnp.float32),
        ),
        interpret=interpret,
    )(xt)

    quantized = q_t.T.reshape(orig_shape)
    i1 = idx1.reshape(orig_shape[:-1])
    i2 = idx2.reshape(orig_shape[:-1])
    err = err_sum[0, 0] / jnp.float32(n * 8)
    return (quantized, i1, i2, err)
